# baseline (speedup 1.0000x reference)
import jax
import jax.numpy as jnp
from jax import lax
from jax.experimental import pallas as pl
from jax.experimental.pallas import tpu as pltpu

BATCH = 4
SEQ = 8192
DIM = 1024
W = 512
REM = SEQ - W


def _pool_body(mask_ref, hs_ref, out_ref, tail_v, rem_v, sem_rows, sem_m, sem_f):
    spec = []
    for b in range(BATCH):
        cp = pltpu.make_async_copy(
            hs_ref.at[b].at[pl.ds(SEQ - 1, 1), :],
            out_ref.at[pl.ds(b, 1), :],
            sem_rows,
        )
        cp.start()
        spec.append(cp)

    mcp = pltpu.make_async_copy(mask_ref.at[:, pl.ds(REM, W)], tail_v, sem_m)
    mcp.start()
    mcp.wait()

    iota_w = lax.broadcasted_iota(jnp.int32, (1, W), 1)
    lastw = []
    for b in range(BATCH):
        row = tail_v[pl.ds(b, 1), :]
        lastw.append(jnp.max(jnp.where(row == 1, iota_w, -1)))

    for cp in spec:
        cp.wait()
    iota_r = lax.broadcasted_iota(jnp.int32, (1, REM), 1)
    for b in range(BATCH):

        @pl.when(lastw[b] != W - 1)
        def _(b=b):
            rcp = pltpu.make_async_copy(
                mask_ref.at[pl.ds(b, 1), pl.ds(0, REM)], rem_v, sem_m
            )
            rcp.start()
            rcp.wait()
            rem_last = jnp.max(jnp.where(rem_v[...] == 1, iota_r, -1))
            true_last = jnp.where(
                lastw[b] >= 0, REM + lastw[b], jnp.maximum(rem_last, 0)
            )
            fcp = pltpu.make_async_copy(
                hs_ref.at[b].at[pl.ds(true_last, 1), :],
                out_ref.at[pl.ds(b, 1), :],
                sem_f,
            )
            fcp.start()
            fcp.wait()


def _pool(mask, hidden_states):
    return pl.pallas_call(
        _pool_body,
        out_shape=jax.ShapeDtypeStruct((BATCH, DIM), jnp.float32),
        in_specs=[
            pl.BlockSpec(memory_space=pl.ANY),
            pl.BlockSpec(memory_space=pl.ANY),
        ],
        out_specs=pl.BlockSpec((BATCH, DIM), lambda: (0, 0)),
        scratch_shapes=[
            pltpu.VMEM((BATCH, W), jnp.int32),
            pltpu.VMEM((1, REM), jnp.int32),
            pltpu.SemaphoreType.DMA,
            pltpu.SemaphoreType.DMA,
            pltpu.SemaphoreType.DMA,
        ],
    )(mask, hidden_states)


def kernel(hidden_states, attention_mask):
    mask = attention_mask.astype(jnp.int32)
    return _pool(mask, hidden_states)
